# baseline (speedup 1.0000x reference)
import functools

import jax
import jax.numpy as jnp
from jax import lax
from jax.experimental import pallas as pl
from jax.experimental.pallas import tpu as pltpu
from jax.experimental.pallas import tpu_sc as plsc

N_NODES = 10000
N_EDGES = 320000
NC = 2
NS = 16
NW = NC * NS
CH = 128
NCHUNK = 80
EPW = NCHUNK * CH
E_PAD = NW * EPW
N_PAD = 10240
RPS = N_PAD // NS
DEG_C = 16

_MESH = plsc.VectorSubcoreMesh(core_axis_name="c", subcore_axis_name="s")
_SC_PARAMS = pltpu.CompilerParams(use_tc_tiling_on_sc=False)


@functools.partial(
    pl.kernel,
    mesh=_MESH,
    compiler_params=_SC_PARAMS,
    out_type=jax.ShapeDtypeStruct((NC, N_PAD, DEG_C), jnp.float32),
    scratch_types=[
        pltpu.VMEM((NCHUNK, CH), jnp.int32),
        pltpu.VMEM((CH, DEG_C), jnp.float32),
        pltpu.VMEM_SHARED((N_PAD, DEG_C), jnp.float32),
    ],
)
def _deg_sc(dst_hbm, z_hbm, out_hbm, dst_v, ones_v, acc_sh):
    cid = lax.axis_index("c")
    sid = lax.axis_index("s")
    wid = sid * NC + cid
    pltpu.sync_copy(z_hbm.at[pl.ds(sid * RPS, RPS)],
                    acc_sh.at[pl.ds(sid * RPS, RPS)])
    pltpu.sync_copy(dst_hbm.at[wid], dst_v)
    for i in range(CH):
        ones_v[i] = jnp.ones((DEG_C,), jnp.float32)
    plsc.subcore_barrier()

    def body(j, carry):
        pltpu.sync_copy(ones_v, acc_sh.at[dst_v.at[j]], add=True)
        return carry

    lax.fori_loop(0, NCHUNK, body, 0)
    plsc.subcore_barrier()
    pltpu.sync_copy(acc_sh.at[pl.ds(sid * RPS, RPS)],
                    out_hbm.at[cid, pl.ds(sid * RPS, RPS)])


def _make_agg_sc(C, NBUF, FULL_IDX):
    NSTAGE = NCHUNK if FULL_IDX else NCHUNK // 2
    steps = NSTAGE // NBUF

    @functools.partial(
        pl.kernel,
        mesh=_MESH,
        compiler_params=_SC_PARAMS,
        out_type=jax.ShapeDtypeStruct((NC, N_PAD, C), jnp.float32),
        scratch_types=(
            [pltpu.VMEM((NSTAGE, CH), jnp.int32),
             pltpu.VMEM((NSTAGE, CH), jnp.int32)]
            + [pltpu.VMEM((CH, C), jnp.float32) for _ in range(NBUF)]
            + [pltpu.VMEM_SHARED((N_PAD, C), jnp.float32)]
            + [pltpu.SemaphoreType.DMA for _ in range(2 * NBUF + 1)]
        ),
    )
    def agg(src_hbm, dst_hbm, g_hbm, out_hbm, src_v, dst_v, *rest):
        rows = rest[:NBUF]
        acc_sh = rest[NBUF]
        gsems = rest[NBUF + 1:2 * NBUF + 1]
        ssems = rest[2 * NBUF + 1:3 * NBUF + 1]
        seedsem = rest[3 * NBUF + 1]
        cid = lax.axis_index("c")
        sid = lax.axis_index("s")
        wid = sid * NC + cid
        last = N_NODES - (NS - 1) * RPS

        @pl.when(sid < NS - 1)
        def _():
            pltpu.async_copy(g_hbm.at[pl.ds(sid * RPS, RPS)],
                             acc_sh.at[pl.ds(sid * RPS, RPS)], seedsem)

        @pl.when(sid == NS - 1)
        def _():
            pltpu.async_copy(g_hbm.at[pl.ds((NS - 1) * RPS, last)],
                             acc_sh.at[pl.ds((NS - 1) * RPS, last)], seedsem)

        pltpu.sync_copy(src_hbm.at[wid, pl.ds(0, NSTAGE)], src_v)
        pltpu.sync_copy(dst_hbm.at[wid, pl.ds(0, NSTAGE)], dst_v)

        @pl.when(sid < NS - 1)
        def _():
            pltpu.make_async_copy(g_hbm.at[pl.ds(sid * RPS, RPS)],
                                  acc_sh.at[pl.ds(sid * RPS, RPS)],
                                  seedsem).wait()

        @pl.when(sid == NS - 1)
        def _():
            pltpu.make_async_copy(g_hbm.at[pl.ds((NS - 1) * RPS, last)],
                                  acc_sh.at[pl.ds((NS - 1) * RPS, last)],
                                  seedsem).wait()

        plsc.subcore_barrier()

        for h in range(1 if FULL_IDX else 2):
            if h:
                pltpu.sync_copy(src_hbm.at[wid, pl.ds(NSTAGE, NSTAGE)], src_v)
                pltpu.sync_copy(dst_hbm.at[wid, pl.ds(NSTAGE, NSTAGE)], dst_v)
            for b in range(NBUF):
                pltpu.async_copy(g_hbm.at[src_v.at[b]], rows[b], gsems[b])

            def body(t, carry):
                for b in range(NBUF):
                    j = NBUF * t + b
                    pltpu.make_async_copy(g_hbm.at[src_v.at[j]], rows[b],
                                          gsems[b]).wait()
                    pltpu.async_copy(rows[b], acc_sh.at[dst_v.at[j]],
                                     ssems[b], add=True)

                    @pl.when(t < steps - 1)
                    def _():
                        pltpu.make_async_copy(rows[b],
                                              acc_sh.at[dst_v.at[j]],
                                              ssems[b]).wait()
                        pltpu.async_copy(g_hbm.at[src_v.at[j + NBUF]],
                                         rows[b], gsems[b])
                return carry

            lax.fori_loop(0, steps, body, 0)
            for b in range(NBUF):
                pltpu.make_async_copy(rows[b], acc_sh.at[dst_v.at[0]],
                                      ssems[b]).wait()
        plsc.subcore_barrier()

        @pl.when(sid < NS - 1)
        def _():
            pltpu.sync_copy(acc_sh.at[pl.ds(sid * RPS, RPS)],
                            out_hbm.at[cid, pl.ds(sid * RPS, RPS)])

        @pl.when(sid == NS - 1)
        def _():
            pltpu.sync_copy(acc_sh.at[pl.ds((NS - 1) * RPS, last)],
                            out_hbm.at[cid, pl.ds((NS - 1) * RPS, last)])

    return agg


_agg128 = _make_agg_sc(128, 2, False)
_agg64 = _make_agg_sc(64, 8, True)
_agg32 = _make_agg_sc(32, 8, True)


BLK = 1000
GRID = N_NODES // BLK


def _tc_mm_body(x_ref, w_ref, out_ref):
    out_ref[...] = jnp.dot(x_ref[...], w_ref[...],
                           preferred_element_type=jnp.float32)


def _tc_mm(x, W1):
    return pl.pallas_call(
        _tc_mm_body,
        grid=(GRID,),
        in_specs=[
            pl.BlockSpec((BLK, 128), lambda i: (i, 0)),
            pl.BlockSpec((128, 128), lambda i: (0, 0)),
        ],
        out_specs=pl.BlockSpec((BLK, 128), lambda i: (i, 0)),
        out_shape=jax.ShapeDtypeStruct((N_NODES, 128), jnp.float32),
    )(x, W1)


def _tc_scale_body(degp_ref, h_ref, d_ref, g_ref):
    deg = degp_ref[0, :, 0:1] + degp_ref[1, :, 0:1] + 1.0
    d = lax.rsqrt(deg)
    d_ref[...] = d
    g_ref[...] = d * h_ref[...]


def _tc_scale(degp, h):
    return pl.pallas_call(
        _tc_scale_body,
        grid=(GRID,),
        in_specs=[
            pl.BlockSpec((NC, BLK, DEG_C), lambda i: (0, i, 0)),
            pl.BlockSpec((BLK, 128), lambda i: (i, 0)),
        ],
        out_specs=[
            pl.BlockSpec((BLK, 1), lambda i: (i, 0)),
            pl.BlockSpec((BLK, 128), lambda i: (i, 0)),
        ],
        out_shape=[
            jax.ShapeDtypeStruct((N_NODES, 1), jnp.float32),
            jax.ShapeDtypeStruct((N_NODES, 128), jnp.float32),
        ],
    )(degp, h)


def _tc_mid_body(aggp_ref, g_ref, d_ref, b_ref, w_ref, out_ref):
    d = d_ref[...]
    h = jnp.maximum(d * (aggp_ref[0] + aggp_ref[1] - g_ref[...]) + b_ref[...],
                    0.0)
    out_ref[...] = d * jnp.dot(h, w_ref[...],
                               preferred_element_type=jnp.float32)


def _make_tc_mid(Cin, Cout):
    def run(aggp, g, d, b, W):
        return pl.pallas_call(
            _tc_mid_body,
            grid=(GRID,),
            in_specs=[
                pl.BlockSpec((NC, BLK, Cin), lambda i: (0, i, 0)),
                pl.BlockSpec((BLK, Cin), lambda i: (i, 0)),
                pl.BlockSpec((BLK, 1), lambda i: (i, 0)),
                pl.BlockSpec((1, Cin), lambda i: (0, 0)),
                pl.BlockSpec((Cin, Cout), lambda i: (0, 0)),
            ],
            out_specs=pl.BlockSpec((BLK, Cout), lambda i: (i, 0)),
            out_shape=jax.ShapeDtypeStruct((N_NODES, Cout), jnp.float32),
        )(aggp, g, d, b, W)

    return run


_tc_mid_128_64 = _make_tc_mid(128, 64)
_tc_mid_64_32 = _make_tc_mid(64, 32)


def _tc_final_body(aggp_ref, g_ref, d_ref, b_ref, wl_ref, bl_ref, out_ref):
    d = d_ref[...]
    h = jnp.maximum(d * (aggp_ref[0] + aggp_ref[1] - g_ref[...]) + b_ref[...],
                    0.0)
    out_ref[...] = jnp.dot(h, wl_ref[...],
                           preferred_element_type=jnp.float32) + bl_ref[...]


def _tc_final(aggp, g, d, b, Wl, bl):
    return pl.pallas_call(
        _tc_final_body,
        grid=(GRID,),
        in_specs=[
            pl.BlockSpec((NC, BLK, 32), lambda i: (0, i, 0)),
            pl.BlockSpec((BLK, 32), lambda i: (i, 0)),
            pl.BlockSpec((BLK, 1), lambda i: (i, 0)),
            pl.BlockSpec((1, 32), lambda i: (0, 0)),
            pl.BlockSpec((32, 1), lambda i: (0, 0)),
            pl.BlockSpec((1, 1), lambda i: (0, 0)),
        ],
        out_specs=pl.BlockSpec((BLK, 1), lambda i: (i, 0)),
        out_shape=jax.ShapeDtypeStruct((N_NODES, 1), jnp.float32),
    )(aggp, g, d, b, Wl, bl)


def kernel(x, edge_index, W1, b1, W2, b2, W3, b3, Wl, bl):
    ei = edge_index.astype(jnp.int32)
    npad = E_PAD - N_EDGES
    pad_iota = lax.iota(jnp.int32, npad)
    src3 = jnp.concatenate([ei[0], pad_iota % N_NODES]).reshape(NW, NCHUNK, CH)
    dst3 = jnp.concatenate(
        [ei[1], N_NODES + pad_iota % (N_PAD - N_NODES)]
    ).reshape(NW, NCHUNK, CH)
    z16 = jnp.zeros((N_PAD, DEG_C), jnp.float32)

    h1x = _tc_mm(x, W1)
    degp = _deg_sc(dst3, z16)
    d, g1 = _tc_scale(degp, h1x)
    aggp1 = _agg128(src3, dst3, g1)
    g2 = _tc_mid_128_64(aggp1, g1, d, b1.reshape(1, -1), W2)
    aggp2 = _agg64(src3, dst3, g2)
    g3 = _tc_mid_64_32(aggp2, g2, d, b2.reshape(1, -1), W3)
    aggp3 = _agg32(src3, dst3, g3)
    return _tc_final(aggp3, g3, d, b3.reshape(1, -1), Wl, bl.reshape(1, 1))


# speedup vs baseline: 1.0353x; 1.0353x over previous
import functools

import jax
import jax.numpy as jnp
from jax import lax
from jax.experimental import pallas as pl
from jax.experimental.pallas import tpu as pltpu
from jax.experimental.pallas import tpu_sc as plsc

N_NODES = 10000
N_EDGES = 320000
NC = 2
NS = 16
NW = NC * NS
CH = 128
NCHUNK = 80
EPW = NCHUNK * CH
E_PAD = NW * EPW
N_PAD = 10240
RPS = N_PAD // NS
DEG_C = 16

_MESH = plsc.VectorSubcoreMesh(core_axis_name="c", subcore_axis_name="s")
_SC_PARAMS = pltpu.CompilerParams(use_tc_tiling_on_sc=False)


@functools.partial(
    pl.kernel,
    mesh=_MESH,
    compiler_params=_SC_PARAMS,
    out_type=jax.ShapeDtypeStruct((NC, N_PAD, DEG_C), jnp.float32),
    scratch_types=[
        pltpu.VMEM((NCHUNK, CH), jnp.int32),
        pltpu.VMEM((CH, DEG_C), jnp.float32),
        pltpu.VMEM_SHARED((N_PAD, DEG_C), jnp.float32),
    ],
)
def _deg_sc(dst_hbm, z_hbm, out_hbm, dst_v, ones_v, acc_sh):
    cid = lax.axis_index("c")
    sid = lax.axis_index("s")
    wid = sid * NC + cid
    pltpu.sync_copy(z_hbm.at[pl.ds(sid * RPS, RPS)],
                    acc_sh.at[pl.ds(sid * RPS, RPS)])
    pltpu.sync_copy(dst_hbm.at[wid], dst_v)
    for i in range(CH):
        ones_v[i] = jnp.ones((DEG_C,), jnp.float32)
    plsc.subcore_barrier()

    def body(j, carry):
        pltpu.sync_copy(ones_v, acc_sh.at[dst_v.at[j]], add=True)
        return carry

    lax.fori_loop(0, NCHUNK, body, 0)
    plsc.subcore_barrier()
    pltpu.sync_copy(acc_sh.at[pl.ds(sid * RPS, RPS)],
                    out_hbm.at[cid, pl.ds(sid * RPS, RPS)])


def _make_agg_sc(C, NBUF, FULL_IDX):
    NSTAGE = NCHUNK if FULL_IDX else NCHUNK // 2
    steps = NSTAGE // NBUF

    @functools.partial(
        pl.kernel,
        mesh=_MESH,
        compiler_params=_SC_PARAMS,
        out_type=jax.ShapeDtypeStruct((NC, N_PAD, C), jnp.float32),
        scratch_types=(
            [pltpu.VMEM((NSTAGE, CH), jnp.int32),
             pltpu.VMEM((NSTAGE, CH), jnp.int32)]
            + [pltpu.VMEM((CH, C), jnp.float32) for _ in range(NBUF)]
            + [pltpu.VMEM_SHARED((N_PAD, C), jnp.float32)]
            + [pltpu.SemaphoreType.DMA for _ in range(2 * NBUF + 1)]
        ),
    )
    def agg(src_hbm, dst_hbm, g_hbm, out_hbm, src_v, dst_v, *rest):
        rows = rest[:NBUF]
        acc_sh = rest[NBUF]
        gsems = rest[NBUF + 1:2 * NBUF + 1]
        ssems = rest[2 * NBUF + 1:3 * NBUF + 1]
        seedsem = rest[3 * NBUF + 1]
        cid = lax.axis_index("c")
        sid = lax.axis_index("s")
        wid = sid * NC + cid
        last = N_NODES - (NS - 1) * RPS

        @pl.when(sid < NS - 1)
        def _():
            pltpu.async_copy(g_hbm.at[pl.ds(sid * RPS, RPS)],
                             acc_sh.at[pl.ds(sid * RPS, RPS)], seedsem)

        @pl.when(sid == NS - 1)
        def _():
            pltpu.async_copy(g_hbm.at[pl.ds((NS - 1) * RPS, last)],
                             acc_sh.at[pl.ds((NS - 1) * RPS, last)], seedsem)

        pltpu.sync_copy(src_hbm.at[wid, pl.ds(0, NSTAGE)], src_v)
        pltpu.sync_copy(dst_hbm.at[wid, pl.ds(0, NSTAGE)], dst_v)

        @pl.when(sid < NS - 1)
        def _():
            pltpu.make_async_copy(g_hbm.at[pl.ds(sid * RPS, RPS)],
                                  acc_sh.at[pl.ds(sid * RPS, RPS)],
                                  seedsem).wait()

        @pl.when(sid == NS - 1)
        def _():
            pltpu.make_async_copy(g_hbm.at[pl.ds((NS - 1) * RPS, last)],
                                  acc_sh.at[pl.ds((NS - 1) * RPS, last)],
                                  seedsem).wait()

        plsc.subcore_barrier()

        for h in range(1 if FULL_IDX else 2):
            if h:
                pltpu.sync_copy(src_hbm.at[wid, pl.ds(NSTAGE, NSTAGE)], src_v)
                pltpu.sync_copy(dst_hbm.at[wid, pl.ds(NSTAGE, NSTAGE)], dst_v)
            for b in range(NBUF):
                pltpu.async_copy(g_hbm.at[src_v.at[b]], rows[b], gsems[b])

            def body(t, carry):
                for b in range(NBUF):
                    j = NBUF * t + b
                    pltpu.make_async_copy(g_hbm.at[src_v.at[j]], rows[b],
                                          gsems[b]).wait()
                    pltpu.async_copy(rows[b], acc_sh.at[dst_v.at[j]],
                                     ssems[b], add=True)

                    @pl.when(t < steps - 1)
                    def _():
                        pltpu.make_async_copy(rows[b],
                                              acc_sh.at[dst_v.at[j]],
                                              ssems[b]).wait()
                        pltpu.async_copy(g_hbm.at[src_v.at[j + NBUF]],
                                         rows[b], gsems[b])
                return carry

            lax.fori_loop(0, steps, body, 0)
            for b in range(NBUF):
                pltpu.make_async_copy(rows[b], acc_sh.at[dst_v.at[0]],
                                      ssems[b]).wait()
        plsc.subcore_barrier()

        @pl.when(sid < NS - 1)
        def _():
            pltpu.sync_copy(acc_sh.at[pl.ds(sid * RPS, RPS)],
                            out_hbm.at[cid, pl.ds(sid * RPS, RPS)])

        @pl.when(sid == NS - 1)
        def _():
            pltpu.sync_copy(acc_sh.at[pl.ds((NS - 1) * RPS, last)],
                            out_hbm.at[cid, pl.ds((NS - 1) * RPS, last)])

    return agg


C96 = 96
NCK96 = 108
E_PAD96 = NW * NCK96 * C96


def _make_agg128_96(NBUF=3):
    NSTAGE = NCK96 // 2
    steps = NSTAGE // NBUF

    @functools.partial(
        pl.kernel,
        mesh=_MESH,
        compiler_params=_SC_PARAMS,
        out_type=jax.ShapeDtypeStruct((NC, N_PAD, 128), jnp.float32),
        scratch_types=(
            [pltpu.VMEM((NSTAGE, C96), jnp.int32),
             pltpu.VMEM((NSTAGE, C96), jnp.int32)]
            + [pltpu.VMEM((C96, 128), jnp.float32) for _ in range(NBUF)]
            + [pltpu.VMEM_SHARED((N_PAD, 128), jnp.float32)]
            + [pltpu.SemaphoreType.DMA for _ in range(2 * NBUF + 1)]
        ),
    )
    def agg(src_hbm, dst_hbm, g_hbm, out_hbm, src_v, dst_v, *rest):
        rows = rest[:NBUF]
        acc_sh = rest[NBUF]
        gsems = rest[NBUF + 1:2 * NBUF + 1]
        ssems = rest[2 * NBUF + 1:3 * NBUF + 1]
        seedsem = rest[3 * NBUF + 1]
        cid = lax.axis_index("c")
        sid = lax.axis_index("s")
        wid = sid * NC + cid
        last = N_NODES - (NS - 1) * RPS

        @pl.when(sid < NS - 1)
        def _():
            pltpu.async_copy(g_hbm.at[pl.ds(sid * RPS, RPS)],
                             acc_sh.at[pl.ds(sid * RPS, RPS)], seedsem)

        @pl.when(sid == NS - 1)
        def _():
            pltpu.async_copy(g_hbm.at[pl.ds((NS - 1) * RPS, last)],
                             acc_sh.at[pl.ds((NS - 1) * RPS, last)], seedsem)

        pltpu.sync_copy(src_hbm.at[wid, pl.ds(0, NSTAGE)], src_v)
        pltpu.sync_copy(dst_hbm.at[wid, pl.ds(0, NSTAGE)], dst_v)

        @pl.when(sid < NS - 1)
        def _():
            pltpu.make_async_copy(g_hbm.at[pl.ds(sid * RPS, RPS)],
                                  acc_sh.at[pl.ds(sid * RPS, RPS)],
                                  seedsem).wait()

        @pl.when(sid == NS - 1)
        def _():
            pltpu.make_async_copy(g_hbm.at[pl.ds((NS - 1) * RPS, last)],
                                  acc_sh.at[pl.ds((NS - 1) * RPS, last)],
                                  seedsem).wait()

        plsc.subcore_barrier()

        for h in range(2):
            if h:
                pltpu.sync_copy(src_hbm.at[wid, pl.ds(NSTAGE, NSTAGE)], src_v)
                pltpu.sync_copy(dst_hbm.at[wid, pl.ds(NSTAGE, NSTAGE)], dst_v)
            for b in range(NBUF):
                pltpu.async_copy(g_hbm.at[src_v.at[b]], rows[b], gsems[b])

            def body(t, carry):
                for b in range(NBUF):
                    j = NBUF * t + b
                    pltpu.make_async_copy(g_hbm.at[src_v.at[j]], rows[b],
                                          gsems[b]).wait()
                    pltpu.async_copy(rows[b], acc_sh.at[dst_v.at[j]],
                                     ssems[b], add=True)

                    @pl.when(t < steps - 1)
                    def _():
                        pltpu.make_async_copy(rows[b],
                                              acc_sh.at[dst_v.at[j]],
                                              ssems[b]).wait()
                        pltpu.async_copy(g_hbm.at[src_v.at[j + NBUF]],
                                         rows[b], gsems[b])
                return carry

            lax.fori_loop(0, steps, body, 0)
            for b in range(NBUF):
                pltpu.make_async_copy(rows[b], acc_sh.at[dst_v.at[0]],
                                      ssems[b]).wait()
        plsc.subcore_barrier()

        @pl.when(sid < NS - 1)
        def _():
            pltpu.sync_copy(acc_sh.at[pl.ds(sid * RPS, RPS)],
                            out_hbm.at[cid, pl.ds(sid * RPS, RPS)])

        @pl.when(sid == NS - 1)
        def _():
            pltpu.sync_copy(acc_sh.at[pl.ds((NS - 1) * RPS, last)],
                            out_hbm.at[cid, pl.ds((NS - 1) * RPS, last)])

    return agg


_agg128_96 = _make_agg128_96()

_agg64 = _make_agg_sc(64, 4, True)
_agg32 = _make_agg_sc(32, 8, True)


BLK = 1000
GRID = N_NODES // BLK


def _tc_mm_body(x_ref, w_ref, out_ref):
    out_ref[...] = jnp.dot(x_ref[...], w_ref[...],
                           preferred_element_type=jnp.float32)


def _tc_mm(x, W1):
    return pl.pallas_call(
        _tc_mm_body,
        grid=(GRID,),
        in_specs=[
            pl.BlockSpec((BLK, 128), lambda i: (i, 0)),
            pl.BlockSpec((128, 128), lambda i: (0, 0)),
        ],
        out_specs=pl.BlockSpec((BLK, 128), lambda i: (i, 0)),
        out_shape=jax.ShapeDtypeStruct((N_NODES, 128), jnp.float32),
    )(x, W1)


def _tc_scale_body(degp_ref, h_ref, d_ref, g_ref):
    deg = degp_ref[0, :, 0:1] + degp_ref[1, :, 0:1] + 1.0
    d = lax.rsqrt(deg)
    d_ref[...] = d
    g_ref[...] = d * h_ref[...]


def _tc_scale(degp, h):
    return pl.pallas_call(
        _tc_scale_body,
        grid=(GRID,),
        in_specs=[
            pl.BlockSpec((NC, BLK, DEG_C), lambda i: (0, i, 0)),
            pl.BlockSpec((BLK, 128), lambda i: (i, 0)),
        ],
        out_specs=[
            pl.BlockSpec((BLK, 1), lambda i: (i, 0)),
            pl.BlockSpec((BLK, 128), lambda i: (i, 0)),
        ],
        out_shape=[
            jax.ShapeDtypeStruct((N_NODES, 1), jnp.float32),
            jax.ShapeDtypeStruct((N_NODES, 128), jnp.float32),
        ],
    )(degp, h)


def _tc_mid_body(aggp_ref, g_ref, d_ref, b_ref, w_ref, out_ref):
    d = d_ref[...]
    h = jnp.maximum(d * (aggp_ref[0] + aggp_ref[1] - g_ref[...]) + b_ref[...],
                    0.0)
    out_ref[...] = d * jnp.dot(h, w_ref[...],
                               preferred_element_type=jnp.float32)


def _make_tc_mid(Cin, Cout):
    def run(aggp, g, d, b, W):
        return pl.pallas_call(
            _tc_mid_body,
            grid=(GRID,),
            in_specs=[
                pl.BlockSpec((NC, BLK, Cin), lambda i: (0, i, 0)),
                pl.BlockSpec((BLK, Cin), lambda i: (i, 0)),
                pl.BlockSpec((BLK, 1), lambda i: (i, 0)),
                pl.BlockSpec((1, Cin), lambda i: (0, 0)),
                pl.BlockSpec((Cin, Cout), lambda i: (0, 0)),
            ],
            out_specs=pl.BlockSpec((BLK, Cout), lambda i: (i, 0)),
            out_shape=jax.ShapeDtypeStruct((N_NODES, Cout), jnp.float32),
        )(aggp, g, d, b, W)

    return run


_tc_mid_128_64 = _make_tc_mid(128, 64)
_tc_mid_64_32 = _make_tc_mid(64, 32)


def _tc_final_body(aggp_ref, g_ref, d_ref, b_ref, wl_ref, bl_ref, out_ref):
    d = d_ref[...]
    h = jnp.maximum(d * (aggp_ref[0] + aggp_ref[1] - g_ref[...]) + b_ref[...],
                    0.0)
    out_ref[...] = jnp.dot(h, wl_ref[...],
                           preferred_element_type=jnp.float32) + bl_ref[...]


def _tc_final(aggp, g, d, b, Wl, bl):
    return pl.pallas_call(
        _tc_final_body,
        grid=(GRID,),
        in_specs=[
            pl.BlockSpec((NC, BLK, 32), lambda i: (0, i, 0)),
            pl.BlockSpec((BLK, 32), lambda i: (i, 0)),
            pl.BlockSpec((BLK, 1), lambda i: (i, 0)),
            pl.BlockSpec((1, 32), lambda i: (0, 0)),
            pl.BlockSpec((32, 1), lambda i: (0, 0)),
            pl.BlockSpec((1, 1), lambda i: (0, 0)),
        ],
        out_specs=pl.BlockSpec((BLK, 1), lambda i: (i, 0)),
        out_shape=jax.ShapeDtypeStruct((N_NODES, 1), jnp.float32),
    )(aggp, g, d, b, Wl, bl)


def kernel(x, edge_index, W1, b1, W2, b2, W3, b3, Wl, bl):
    ei = edge_index.astype(jnp.int32)
    npad = E_PAD - N_EDGES
    pad_iota = lax.iota(jnp.int32, npad)
    src3 = jnp.concatenate([ei[0], pad_iota % N_NODES]).reshape(NW, NCHUNK, CH)
    dst3 = jnp.concatenate(
        [ei[1], N_NODES + pad_iota % (N_PAD - N_NODES)]
    ).reshape(NW, NCHUNK, CH)
    pad96 = lax.iota(jnp.int32, E_PAD96 - N_EDGES)
    src96 = jnp.concatenate([ei[0], pad96 % N_NODES]).reshape(NW, NCK96, C96)
    dst96 = jnp.concatenate(
        [ei[1], N_NODES + pad96 % (N_PAD - N_NODES)]
    ).reshape(NW, NCK96, C96)
    z16 = jnp.zeros((N_PAD, DEG_C), jnp.float32)

    h1x = _tc_mm(x, W1)
    degp = _deg_sc(dst3, z16)
    d, g1 = _tc_scale(degp, h1x)
    aggp1 = _agg128_96(src96, dst96, g1)
    g2 = _tc_mid_128_64(aggp1, g1, d, b1.reshape(1, -1), W2)
    aggp2 = _agg64(src3, dst3, g2)
    g3 = _tc_mid_64_32(aggp2, g2, d, b2.reshape(1, -1), W3)
    aggp3 = _agg32(src3, dst3, g3)
    return _tc_final(aggp3, g3, d, b3.reshape(1, -1), Wl, bl.reshape(1, 1))


# speedup vs baseline: 1.0475x; 1.0118x over previous
import functools

import jax
import jax.numpy as jnp
from jax import lax
from jax.experimental import pallas as pl
from jax.experimental.pallas import tpu as pltpu
from jax.experimental.pallas import tpu_sc as plsc

N_NODES = 10000
N_EDGES = 320000
NC = 2
NS = 16
NW = NC * NS
CH = 128
NCHUNK = 80
EPW = NCHUNK * CH
E_PAD = NW * EPW
N_PAD = 10240
RPS = N_PAD // NS
DEG_C = 16

_MESH = plsc.VectorSubcoreMesh(core_axis_name="c", subcore_axis_name="s")
_SC_PARAMS = pltpu.CompilerParams(use_tc_tiling_on_sc=False)


@functools.partial(
    pl.kernel,
    mesh=_MESH,
    compiler_params=_SC_PARAMS,
    out_type=jax.ShapeDtypeStruct((NC, N_PAD, DEG_C), jnp.float32),
    scratch_types=[
        pltpu.VMEM((NCHUNK, CH), jnp.int32),
        pltpu.VMEM((CH, DEG_C), jnp.float32),
        pltpu.VMEM_SHARED((N_PAD, DEG_C), jnp.float32),
        pltpu.SemaphoreType.DMA,
        pltpu.SemaphoreType.DMA,
        pltpu.SemaphoreType.DMA,
        pltpu.SemaphoreType.DMA,
    ],
)
def _deg_sc(dst_hbm, z_hbm, out_hbm, dst_v, ones_v, acc_sh, *ssems):
    cid = lax.axis_index("c")
    sid = lax.axis_index("s")
    wid = sid * NC + cid
    pltpu.sync_copy(z_hbm.at[pl.ds(sid * RPS, RPS)],
                    acc_sh.at[pl.ds(sid * RPS, RPS)])
    pltpu.sync_copy(dst_hbm.at[wid], dst_v)
    for i in range(CH):
        ones_v[i] = jnp.ones((DEG_C,), jnp.float32)
    plsc.subcore_barrier()

    def body(t, carry):
        for b in range(4):
            j = 4 * t + b

            @pl.when(t > 0)
            def _():
                pltpu.make_async_copy(ones_v, acc_sh.at[dst_v.at[j]],
                                      ssems[b]).wait()

            pltpu.async_copy(ones_v, acc_sh.at[dst_v.at[j]],
                             ssems[b], add=True)
        return carry

    lax.fori_loop(0, NCHUNK // 4, body, 0)
    for b in range(4):
        pltpu.make_async_copy(ones_v, acc_sh.at[dst_v.at[0]],
                              ssems[b]).wait()
    plsc.subcore_barrier()
    pltpu.sync_copy(acc_sh.at[pl.ds(sid * RPS, RPS)],
                    out_hbm.at[cid, pl.ds(sid * RPS, RPS)])


def _make_agg_sc(C, NBUF, FULL_IDX):
    NSTAGE = NCHUNK if FULL_IDX else NCHUNK // 2
    steps = NSTAGE // NBUF

    @functools.partial(
        pl.kernel,
        mesh=_MESH,
        compiler_params=_SC_PARAMS,
        out_type=jax.ShapeDtypeStruct((NC, N_PAD, C), jnp.float32),
        scratch_types=(
            [pltpu.VMEM((NSTAGE, CH), jnp.int32),
             pltpu.VMEM((NSTAGE, CH), jnp.int32)]
            + [pltpu.VMEM((CH, C), jnp.float32) for _ in range(NBUF)]
            + [pltpu.VMEM_SHARED((N_PAD, C), jnp.float32)]
            + [pltpu.SemaphoreType.DMA for _ in range(2 * NBUF + 1)]
        ),
    )
    def agg(src_hbm, dst_hbm, g_hbm, out_hbm, src_v, dst_v, *rest):
        rows = rest[:NBUF]
        acc_sh = rest[NBUF]
        gsems = rest[NBUF + 1:2 * NBUF + 1]
        ssems = rest[2 * NBUF + 1:3 * NBUF + 1]
        seedsem = rest[3 * NBUF + 1]
        cid = lax.axis_index("c")
        sid = lax.axis_index("s")
        wid = sid * NC + cid
        last = N_NODES - (NS - 1) * RPS

        @pl.when(sid < NS - 1)
        def _():
            pltpu.async_copy(g_hbm.at[pl.ds(sid * RPS, RPS)],
                             acc_sh.at[pl.ds(sid * RPS, RPS)], seedsem)

        @pl.when(sid == NS - 1)
        def _():
            pltpu.async_copy(g_hbm.at[pl.ds((NS - 1) * RPS, last)],
                             acc_sh.at[pl.ds((NS - 1) * RPS, last)], seedsem)

        pltpu.sync_copy(src_hbm.at[wid, pl.ds(0, NSTAGE)], src_v)
        pltpu.sync_copy(dst_hbm.at[wid, pl.ds(0, NSTAGE)], dst_v)

        @pl.when(sid < NS - 1)
        def _():
            pltpu.make_async_copy(g_hbm.at[pl.ds(sid * RPS, RPS)],
                                  acc_sh.at[pl.ds(sid * RPS, RPS)],
                                  seedsem).wait()

        @pl.when(sid == NS - 1)
        def _():
            pltpu.make_async_copy(g_hbm.at[pl.ds((NS - 1) * RPS, last)],
                                  acc_sh.at[pl.ds((NS - 1) * RPS, last)],
                                  seedsem).wait()

        plsc.subcore_barrier()

        for h in range(1 if FULL_IDX else 2):
            if h:
                pltpu.sync_copy(src_hbm.at[wid, pl.ds(NSTAGE, NSTAGE)], src_v)
                pltpu.sync_copy(dst_hbm.at[wid, pl.ds(NSTAGE, NSTAGE)], dst_v)
            for b in range(NBUF):
                pltpu.async_copy(g_hbm.at[src_v.at[b]], rows[b], gsems[b])

            def body(t, carry):
                for b in range(NBUF):
                    j = NBUF * t + b
                    pltpu.make_async_copy(g_hbm.at[src_v.at[j]], rows[b],
                                          gsems[b]).wait()
                    pltpu.async_copy(rows[b], acc_sh.at[dst_v.at[j]],
                                     ssems[b], add=True)

                    @pl.when(t < steps - 1)
                    def _():
                        pltpu.make_async_copy(rows[b],
                                              acc_sh.at[dst_v.at[j]],
                                              ssems[b]).wait()
                        pltpu.async_copy(g_hbm.at[src_v.at[j + NBUF]],
                                         rows[b], gsems[b])
                return carry

            lax.fori_loop(0, steps, body, 0)
            for b in range(NBUF):
                pltpu.make_async_copy(rows[b], acc_sh.at[dst_v.at[0]],
                                      ssems[b]).wait()
        plsc.subcore_barrier()

        @pl.when(sid < NS - 1)
        def _():
            pltpu.sync_copy(acc_sh.at[pl.ds(sid * RPS, RPS)],
                            out_hbm.at[cid, pl.ds(sid * RPS, RPS)])

        @pl.when(sid == NS - 1)
        def _():
            pltpu.sync_copy(acc_sh.at[pl.ds((NS - 1) * RPS, last)],
                            out_hbm.at[cid, pl.ds((NS - 1) * RPS, last)])

    return agg


C96 = 96
NCK96 = 108
E_PAD96 = NW * NCK96 * C96


def _make_agg128_96(NBUF=3):
    NSTAGE = NCK96 // 2
    steps = NSTAGE // NBUF

    @functools.partial(
        pl.kernel,
        mesh=_MESH,
        compiler_params=_SC_PARAMS,
        out_type=jax.ShapeDtypeStruct((NC, N_PAD, 128), jnp.float32),
        scratch_types=(
            [pltpu.VMEM((NSTAGE, C96), jnp.int32),
             pltpu.VMEM((NSTAGE, C96), jnp.int32)]
            + [pltpu.VMEM((C96, 128), jnp.float32) for _ in range(NBUF)]
            + [pltpu.VMEM_SHARED((N_PAD, 128), jnp.float32)]
            + [pltpu.SemaphoreType.DMA for _ in range(2 * NBUF + 1)]
        ),
    )
    def agg(src_hbm, dst_hbm, g_hbm, out_hbm, src_v, dst_v, *rest):
        rows = rest[:NBUF]
        acc_sh = rest[NBUF]
        gsems = rest[NBUF + 1:2 * NBUF + 1]
        ssems = rest[2 * NBUF + 1:3 * NBUF + 1]
        seedsem = rest[3 * NBUF + 1]
        cid = lax.axis_index("c")
        sid = lax.axis_index("s")
        wid = sid * NC + cid
        last = N_NODES - (NS - 1) * RPS

        @pl.when(sid < NS - 1)
        def _():
            pltpu.async_copy(g_hbm.at[pl.ds(sid * RPS, RPS)],
                             acc_sh.at[pl.ds(sid * RPS, RPS)], seedsem)

        @pl.when(sid == NS - 1)
        def _():
            pltpu.async_copy(g_hbm.at[pl.ds((NS - 1) * RPS, last)],
                             acc_sh.at[pl.ds((NS - 1) * RPS, last)], seedsem)

        pltpu.sync_copy(src_hbm.at[wid, pl.ds(0, NSTAGE)], src_v)
        pltpu.sync_copy(dst_hbm.at[wid, pl.ds(0, NSTAGE)], dst_v)

        @pl.when(sid < NS - 1)
        def _():
            pltpu.make_async_copy(g_hbm.at[pl.ds(sid * RPS, RPS)],
                                  acc_sh.at[pl.ds(sid * RPS, RPS)],
                                  seedsem).wait()

        @pl.when(sid == NS - 1)
        def _():
            pltpu.make_async_copy(g_hbm.at[pl.ds((NS - 1) * RPS, last)],
                                  acc_sh.at[pl.ds((NS - 1) * RPS, last)],
                                  seedsem).wait()

        plsc.subcore_barrier()

        for h in range(2):
            if h:
                pltpu.sync_copy(src_hbm.at[wid, pl.ds(NSTAGE, NSTAGE)], src_v)
                pltpu.sync_copy(dst_hbm.at[wid, pl.ds(NSTAGE, NSTAGE)], dst_v)
            for b in range(NBUF):
                pltpu.async_copy(g_hbm.at[src_v.at[b]], rows[b], gsems[b])

            def body(t, carry):
                for b in range(NBUF):
                    j = NBUF * t + b
                    pltpu.make_async_copy(g_hbm.at[src_v.at[j]], rows[b],
                                          gsems[b]).wait()
                    pltpu.async_copy(rows[b], acc_sh.at[dst_v.at[j]],
                                     ssems[b], add=True)

                    @pl.when(t < steps - 1)
                    def _():
                        pltpu.make_async_copy(rows[b],
                                              acc_sh.at[dst_v.at[j]],
                                              ssems[b]).wait()
                        pltpu.async_copy(g_hbm.at[src_v.at[j + NBUF]],
                                         rows[b], gsems[b])
                return carry

            lax.fori_loop(0, steps, body, 0)
            for b in range(NBUF):
                pltpu.make_async_copy(rows[b], acc_sh.at[dst_v.at[0]],
                                      ssems[b]).wait()
        plsc.subcore_barrier()

        @pl.when(sid < NS - 1)
        def _():
            pltpu.sync_copy(acc_sh.at[pl.ds(sid * RPS, RPS)],
                            out_hbm.at[cid, pl.ds(sid * RPS, RPS)])

        @pl.when(sid == NS - 1)
        def _():
            pltpu.sync_copy(acc_sh.at[pl.ds((NS - 1) * RPS, last)],
                            out_hbm.at[cid, pl.ds((NS - 1) * RPS, last)])

    return agg


_agg128_96 = _make_agg128_96()

_agg64 = _make_agg_sc(64, 4, True)
_agg32 = _make_agg_sc(32, 8, True)


BLK = 1000
GRID = N_NODES // BLK


def _tc_mm_body(x_ref, w_ref, out_ref):
    out_ref[...] = jnp.dot(x_ref[...], w_ref[...],
                           preferred_element_type=jnp.float32)


def _tc_mm(x, W1):
    return pl.pallas_call(
        _tc_mm_body,
        grid=(GRID,),
        in_specs=[
            pl.BlockSpec((BLK, 128), lambda i: (i, 0)),
            pl.BlockSpec((128, 128), lambda i: (0, 0)),
        ],
        out_specs=pl.BlockSpec((BLK, 128), lambda i: (i, 0)),
        out_shape=jax.ShapeDtypeStruct((N_NODES, 128), jnp.float32),
    )(x, W1)


def _tc_scale_body(degp_ref, h_ref, d_ref, g_ref):
    deg = degp_ref[0, :, 0:1] + degp_ref[1, :, 0:1] + 1.0
    d = lax.rsqrt(deg)
    d_ref[...] = d
    g_ref[...] = d * h_ref[...]


def _tc_scale(degp, h):
    return pl.pallas_call(
        _tc_scale_body,
        grid=(GRID,),
        in_specs=[
            pl.BlockSpec((NC, BLK, DEG_C), lambda i: (0, i, 0)),
            pl.BlockSpec((BLK, 128), lambda i: (i, 0)),
        ],
        out_specs=[
            pl.BlockSpec((BLK, 1), lambda i: (i, 0)),
            pl.BlockSpec((BLK, 128), lambda i: (i, 0)),
        ],
        out_shape=[
            jax.ShapeDtypeStruct((N_NODES, 1), jnp.float32),
            jax.ShapeDtypeStruct((N_NODES, 128), jnp.float32),
        ],
    )(degp, h)


def _tc_mid_body(aggp_ref, g_ref, d_ref, b_ref, w_ref, out_ref):
    d = d_ref[...]
    h = jnp.maximum(d * (aggp_ref[0] + aggp_ref[1] - g_ref[...]) + b_ref[...],
                    0.0)
    out_ref[...] = d * jnp.dot(h, w_ref[...],
                               preferred_element_type=jnp.float32)


def _make_tc_mid(Cin, Cout):
    def run(aggp, g, d, b, W):
        return pl.pallas_call(
            _tc_mid_body,
            grid=(GRID,),
            in_specs=[
                pl.BlockSpec((NC, BLK, Cin), lambda i: (0, i, 0)),
                pl.BlockSpec((BLK, Cin), lambda i: (i, 0)),
                pl.BlockSpec((BLK, 1), lambda i: (i, 0)),
                pl.BlockSpec((1, Cin), lambda i: (0, 0)),
                pl.BlockSpec((Cin, Cout), lambda i: (0, 0)),
            ],
            out_specs=pl.BlockSpec((BLK, Cout), lambda i: (i, 0)),
            out_shape=jax.ShapeDtypeStruct((N_NODES, Cout), jnp.float32),
        )(aggp, g, d, b, W)

    return run


_tc_mid_128_64 = _make_tc_mid(128, 64)
_tc_mid_64_32 = _make_tc_mid(64, 32)


def _tc_final_body(aggp_ref, g_ref, d_ref, b_ref, wl_ref, bl_ref, out_ref):
    d = d_ref[...]
    h = jnp.maximum(d * (aggp_ref[0] + aggp_ref[1] - g_ref[...]) + b_ref[...],
                    0.0)
    out_ref[...] = jnp.dot(h, wl_ref[...],
                           preferred_element_type=jnp.float32) + bl_ref[...]


def _tc_final(aggp, g, d, b, Wl, bl):
    return pl.pallas_call(
        _tc_final_body,
        grid=(GRID,),
        in_specs=[
            pl.BlockSpec((NC, BLK, 32), lambda i: (0, i, 0)),
            pl.BlockSpec((BLK, 32), lambda i: (i, 0)),
            pl.BlockSpec((BLK, 1), lambda i: (i, 0)),
            pl.BlockSpec((1, 32), lambda i: (0, 0)),
            pl.BlockSpec((32, 1), lambda i: (0, 0)),
            pl.BlockSpec((1, 1), lambda i: (0, 0)),
        ],
        out_specs=pl.BlockSpec((BLK, 1), lambda i: (i, 0)),
        out_shape=jax.ShapeDtypeStruct((N_NODES, 1), jnp.float32),
    )(aggp, g, d, b, Wl, bl)


def kernel(x, edge_index, W1, b1, W2, b2, W3, b3, Wl, bl):
    ei = edge_index.astype(jnp.int32)
    npad = E_PAD - N_EDGES
    pad_iota = lax.iota(jnp.int32, npad)
    src3 = jnp.concatenate([ei[0], pad_iota % N_NODES]).reshape(NW, NCHUNK, CH)
    dst3 = jnp.concatenate(
        [ei[1], N_NODES + pad_iota % (N_PAD - N_NODES)]
    ).reshape(NW, NCHUNK, CH)
    pad96 = lax.iota(jnp.int32, E_PAD96 - N_EDGES)
    src96 = jnp.concatenate([ei[0], pad96 % N_NODES]).reshape(NW, NCK96, C96)
    dst96 = jnp.concatenate(
        [ei[1], N_NODES + pad96 % (N_PAD - N_NODES)]
    ).reshape(NW, NCK96, C96)
    z16 = jnp.zeros((N_PAD, DEG_C), jnp.float32)

    h1x = _tc_mm(x, W1)
    degp = _deg_sc(dst3, z16)
    d, g1 = _tc_scale(degp, h1x)
    aggp1 = _agg128_96(src96, dst96, g1)
    g2 = _tc_mid_128_64(aggp1, g1, d, b1.reshape(1, -1), W2)
    aggp2 = _agg64(src3, dst3, g2)
    g3 = _tc_mid_64_32(aggp2, g2, d, b2.reshape(1, -1), W3)
    aggp3 = _agg32(src3, dst3, g3)
    return _tc_final(aggp3, g3, d, b3.reshape(1, -1), Wl, bl.reshape(1, 1))


# speedup vs baseline: 1.0761x; 1.0273x over previous
import functools

import jax
import jax.numpy as jnp
from jax import lax
from jax.experimental import pallas as pl
from jax.experimental.pallas import tpu as pltpu
from jax.experimental.pallas import tpu_sc as plsc

N_NODES = 10000
N_EDGES = 320000
NC = 2
NS = 16
NW = NC * NS
CH = 128
NCHUNK = 80
EPW = NCHUNK * CH
E_PAD = NW * EPW
N_PAD = 10240
RPS = N_PAD // NS
DEG_C = 16

_MESH = plsc.VectorSubcoreMesh(core_axis_name="c", subcore_axis_name="s")
_SC_PARAMS = pltpu.CompilerParams(use_tc_tiling_on_sc=False)


@functools.partial(
    pl.kernel,
    mesh=_MESH,
    compiler_params=_SC_PARAMS,
    out_type=jax.ShapeDtypeStruct((NC, N_PAD, DEG_C), jnp.float32),
    scratch_types=[
        pltpu.VMEM((NCHUNK, CH), jnp.int32),
        pltpu.VMEM((CH, DEG_C), jnp.float32),
        pltpu.VMEM_SHARED((N_PAD, DEG_C), jnp.float32),
        pltpu.SemaphoreType.DMA,
        pltpu.SemaphoreType.DMA,
        pltpu.SemaphoreType.DMA,
        pltpu.SemaphoreType.DMA,
    ],
)
def _deg_sc(dst_hbm, z_hbm, out_hbm, dst_v, ones_v, acc_sh, *ssems):
    cid = lax.axis_index("c")
    sid = lax.axis_index("s")
    wid = sid * NC + cid
    pltpu.sync_copy(z_hbm.at[pl.ds(sid * RPS, RPS)],
                    acc_sh.at[pl.ds(sid * RPS, RPS)])
    pltpu.sync_copy(dst_hbm.at[wid], dst_v)
    for i in range(CH):
        ones_v[i] = jnp.ones((DEG_C,), jnp.float32)
    plsc.subcore_barrier()

    def body(t, carry):
        for b in range(4):
            j = 4 * t + b

            @pl.when(t > 0)
            def _():
                pltpu.make_async_copy(ones_v, acc_sh.at[dst_v.at[j]],
                                      ssems[b]).wait()

            pltpu.async_copy(ones_v, acc_sh.at[dst_v.at[j]],
                             ssems[b], add=True)
        return carry

    lax.fori_loop(0, NCHUNK // 4, body, 0)
    for b in range(4):
        pltpu.make_async_copy(ones_v, acc_sh.at[dst_v.at[0]],
                              ssems[b]).wait()
    plsc.subcore_barrier()
    pltpu.sync_copy(acc_sh.at[pl.ds(sid * RPS, RPS)],
                    out_hbm.at[cid, pl.ds(sid * RPS, RPS)])


def _make_agg_sc(C, NBUF, FULL_IDX):
    NSTAGE = NCHUNK if FULL_IDX else NCHUNK // 2
    steps = NSTAGE // NBUF

    @functools.partial(
        pl.kernel,
        mesh=_MESH,
        compiler_params=_SC_PARAMS,
        out_type=jax.ShapeDtypeStruct((NC, N_PAD, C), jnp.float32),
        scratch_types=(
            [pltpu.VMEM((NSTAGE, CH), jnp.int32),
             pltpu.VMEM((NSTAGE, CH), jnp.int32)]
            + [pltpu.VMEM((CH, C), jnp.float32) for _ in range(NBUF)]
            + [pltpu.VMEM_SHARED((N_PAD, C), jnp.float32)]
            + [pltpu.SemaphoreType.DMA for _ in range(2 * NBUF + 1)]
        ),
    )
    def agg(src_hbm, dst_hbm, g_hbm, out_hbm, src_v, dst_v, *rest):
        rows = rest[:NBUF]
        acc_sh = rest[NBUF]
        gsems = rest[NBUF + 1:2 * NBUF + 1]
        ssems = rest[2 * NBUF + 1:3 * NBUF + 1]
        seedsem = rest[3 * NBUF + 1]
        cid = lax.axis_index("c")
        sid = lax.axis_index("s")
        wid = sid * NC + cid
        last = N_NODES - (NS - 1) * RPS

        @pl.when(sid < NS - 1)
        def _():
            pltpu.async_copy(g_hbm.at[pl.ds(sid * RPS, RPS)],
                             acc_sh.at[pl.ds(sid * RPS, RPS)], seedsem)

        @pl.when(sid == NS - 1)
        def _():
            pltpu.async_copy(g_hbm.at[pl.ds((NS - 1) * RPS, last)],
                             acc_sh.at[pl.ds((NS - 1) * RPS, last)], seedsem)

        pltpu.sync_copy(src_hbm.at[wid, pl.ds(0, NSTAGE)], src_v)
        pltpu.sync_copy(dst_hbm.at[wid, pl.ds(0, NSTAGE)], dst_v)

        @pl.when(sid < NS - 1)
        def _():
            pltpu.make_async_copy(g_hbm.at[pl.ds(sid * RPS, RPS)],
                                  acc_sh.at[pl.ds(sid * RPS, RPS)],
                                  seedsem).wait()

        @pl.when(sid == NS - 1)
        def _():
            pltpu.make_async_copy(g_hbm.at[pl.ds((NS - 1) * RPS, last)],
                                  acc_sh.at[pl.ds((NS - 1) * RPS, last)],
                                  seedsem).wait()

        plsc.subcore_barrier()

        for h in range(1 if FULL_IDX else 2):
            if h:
                pltpu.sync_copy(src_hbm.at[wid, pl.ds(NSTAGE, NSTAGE)], src_v)
                pltpu.sync_copy(dst_hbm.at[wid, pl.ds(NSTAGE, NSTAGE)], dst_v)
            for b in range(NBUF):
                pltpu.async_copy(g_hbm.at[src_v.at[b]], rows[b], gsems[b])

            def body(t, carry):
                for b in range(NBUF):
                    j = NBUF * t + b
                    pltpu.make_async_copy(g_hbm.at[src_v.at[j]], rows[b],
                                          gsems[b]).wait()
                    pltpu.async_copy(rows[b], acc_sh.at[dst_v.at[j]],
                                     ssems[b], add=True)

                    @pl.when(t < steps - 1)
                    def _():
                        pltpu.make_async_copy(rows[b],
                                              acc_sh.at[dst_v.at[j]],
                                              ssems[b]).wait()
                        pltpu.async_copy(g_hbm.at[src_v.at[j + NBUF]],
                                         rows[b], gsems[b])
                return carry

            lax.fori_loop(0, steps, body, 0)
            for b in range(NBUF):
                pltpu.make_async_copy(rows[b], acc_sh.at[dst_v.at[0]],
                                      ssems[b]).wait()
        plsc.subcore_barrier()

        @pl.when(sid < NS - 1)
        def _():
            pltpu.sync_copy(acc_sh.at[pl.ds(sid * RPS, RPS)],
                            out_hbm.at[cid, pl.ds(sid * RPS, RPS)])

        @pl.when(sid == NS - 1)
        def _():
            pltpu.sync_copy(acc_sh.at[pl.ds((NS - 1) * RPS, last)],
                            out_hbm.at[cid, pl.ds((NS - 1) * RPS, last)])

    return agg


C96 = 96
NCK96 = 108
E_PAD96 = NW * NCK96 * C96


def _make_agg128_96(NBUF=3):
    NSTAGE = NCK96 // 2
    steps = NSTAGE // NBUF

    @functools.partial(
        pl.kernel,
        mesh=_MESH,
        compiler_params=_SC_PARAMS,
        out_type=jax.ShapeDtypeStruct((NC, N_PAD, 128), jnp.float32),
        scratch_types=(
            [pltpu.VMEM((NSTAGE, C96), jnp.int32),
             pltpu.VMEM((NSTAGE, C96), jnp.int32)]
            + [pltpu.VMEM((C96, 128), jnp.float32) for _ in range(NBUF)]
            + [pltpu.VMEM_SHARED((N_PAD, 128), jnp.float32)]
            + [pltpu.SemaphoreType.DMA for _ in range(2 * NBUF + 1)]
        ),
    )
    def agg(src_hbm, dst_hbm, g_hbm, out_hbm, src_v, dst_v, *rest):
        rows = rest[:NBUF]
        acc_sh = rest[NBUF]
        gsems = rest[NBUF + 1:2 * NBUF + 1]
        ssems = rest[2 * NBUF + 1:3 * NBUF + 1]
        seedsem = rest[3 * NBUF + 1]
        cid = lax.axis_index("c")
        sid = lax.axis_index("s")
        wid = sid * NC + cid
        last = N_NODES - (NS - 1) * RPS

        @pl.when(sid < NS - 1)
        def _():
            pltpu.async_copy(g_hbm.at[pl.ds(sid * RPS, RPS)],
                             acc_sh.at[pl.ds(sid * RPS, RPS)], seedsem)

        @pl.when(sid == NS - 1)
        def _():
            pltpu.async_copy(g_hbm.at[pl.ds((NS - 1) * RPS, last)],
                             acc_sh.at[pl.ds((NS - 1) * RPS, last)], seedsem)

        pltpu.sync_copy(src_hbm.at[wid, pl.ds(0, NSTAGE)], src_v)
        pltpu.sync_copy(dst_hbm.at[wid, pl.ds(0, NSTAGE)], dst_v)

        @pl.when(sid < NS - 1)
        def _():
            pltpu.make_async_copy(g_hbm.at[pl.ds(sid * RPS, RPS)],
                                  acc_sh.at[pl.ds(sid * RPS, RPS)],
                                  seedsem).wait()

        @pl.when(sid == NS - 1)
        def _():
            pltpu.make_async_copy(g_hbm.at[pl.ds((NS - 1) * RPS, last)],
                                  acc_sh.at[pl.ds((NS - 1) * RPS, last)],
                                  seedsem).wait()

        plsc.subcore_barrier()

        for h in range(2):
            if h:
                pltpu.sync_copy(src_hbm.at[wid, pl.ds(NSTAGE, NSTAGE)], src_v)
                pltpu.sync_copy(dst_hbm.at[wid, pl.ds(NSTAGE, NSTAGE)], dst_v)
            for b in range(NBUF):
                pltpu.async_copy(g_hbm.at[src_v.at[b]], rows[b], gsems[b])

            def body(t, carry):
                for b in range(NBUF):
                    j = NBUF * t + b
                    pltpu.make_async_copy(g_hbm.at[src_v.at[j]], rows[b],
                                          gsems[b]).wait()
                    pltpu.async_copy(rows[b], acc_sh.at[dst_v.at[j]],
                                     ssems[b], add=True)

                    @pl.when(t < steps - 1)
                    def _():
                        pltpu.make_async_copy(rows[b],
                                              acc_sh.at[dst_v.at[j]],
                                              ssems[b]).wait()
                        pltpu.async_copy(g_hbm.at[src_v.at[j + NBUF]],
                                         rows[b], gsems[b])
                return carry

            lax.fori_loop(0, steps, body, 0)
            for b in range(NBUF):
                pltpu.make_async_copy(rows[b], acc_sh.at[dst_v.at[0]],
                                      ssems[b]).wait()
        plsc.subcore_barrier()

        @pl.when(sid < NS - 1)
        def _():
            pltpu.sync_copy(acc_sh.at[pl.ds(sid * RPS, RPS)],
                            out_hbm.at[cid, pl.ds(sid * RPS, RPS)])

        @pl.when(sid == NS - 1)
        def _():
            pltpu.sync_copy(acc_sh.at[pl.ds((NS - 1) * RPS, last)],
                            out_hbm.at[cid, pl.ds((NS - 1) * RPS, last)])

    return agg


_agg128_96 = _make_agg128_96()

_agg64 = _make_agg_sc(64, 4, True)
_agg32 = _make_agg_sc(32, 8, True)


BLK = 2000
GRID = N_NODES // BLK


def _tc_mm_body(x_ref, w_ref, out_ref):
    out_ref[...] = jnp.dot(x_ref[...], w_ref[...],
                           preferred_element_type=jnp.float32)


def _tc_mm(x, W1):
    return pl.pallas_call(
        _tc_mm_body,
        grid=(GRID,),
        in_specs=[
            pl.BlockSpec((BLK, 128), lambda i: (i, 0)),
            pl.BlockSpec((128, 128), lambda i: (0, 0)),
        ],
        out_specs=pl.BlockSpec((BLK, 128), lambda i: (i, 0)),
        out_shape=jax.ShapeDtypeStruct((N_NODES, 128), jnp.float32),
    )(x, W1)


def _tc_scale_body(degp_ref, h_ref, d_ref, g_ref):
    deg = degp_ref[0, :, 0:1] + degp_ref[1, :, 0:1] + 1.0
    d = lax.rsqrt(deg)
    d_ref[...] = d
    g_ref[...] = d * h_ref[...]


def _tc_scale(degp, h):
    return pl.pallas_call(
        _tc_scale_body,
        grid=(GRID,),
        in_specs=[
            pl.BlockSpec((NC, BLK, DEG_C), lambda i: (0, i, 0)),
            pl.BlockSpec((BLK, 128), lambda i: (i, 0)),
        ],
        out_specs=[
            pl.BlockSpec((BLK, 1), lambda i: (i, 0)),
            pl.BlockSpec((BLK, 128), lambda i: (i, 0)),
        ],
        out_shape=[
            jax.ShapeDtypeStruct((N_NODES, 1), jnp.float32),
            jax.ShapeDtypeStruct((N_NODES, 128), jnp.float32),
        ],
    )(degp, h)


def _tc_mid_body(aggp_ref, g_ref, d_ref, b_ref, w_ref, out_ref):
    d = d_ref[...]
    h = jnp.maximum(d * (aggp_ref[0] + aggp_ref[1] - g_ref[...]) + b_ref[...],
                    0.0)
    out_ref[...] = d * jnp.dot(h, w_ref[...],
                               preferred_element_type=jnp.float32)


def _make_tc_mid(Cin, Cout):
    def run(aggp, g, d, b, W):
        return pl.pallas_call(
            _tc_mid_body,
            grid=(GRID,),
            in_specs=[
                pl.BlockSpec((NC, BLK, Cin), lambda i: (0, i, 0)),
                pl.BlockSpec((BLK, Cin), lambda i: (i, 0)),
                pl.BlockSpec((BLK, 1), lambda i: (i, 0)),
                pl.BlockSpec((1, Cin), lambda i: (0, 0)),
                pl.BlockSpec((Cin, Cout), lambda i: (0, 0)),
            ],
            out_specs=pl.BlockSpec((BLK, Cout), lambda i: (i, 0)),
            out_shape=jax.ShapeDtypeStruct((N_NODES, Cout), jnp.float32),
        )(aggp, g, d, b, W)

    return run


_tc_mid_128_64 = _make_tc_mid(128, 64)
_tc_mid_64_32 = _make_tc_mid(64, 32)


def _tc_final_body(aggp_ref, g_ref, d_ref, b_ref, wl_ref, bl_ref, out_ref):
    d = d_ref[...]
    h = jnp.maximum(d * (aggp_ref[0] + aggp_ref[1] - g_ref[...]) + b_ref[...],
                    0.0)
    out_ref[...] = jnp.dot(h, wl_ref[...],
                           preferred_element_type=jnp.float32) + bl_ref[...]


def _tc_final(aggp, g, d, b, Wl, bl):
    return pl.pallas_call(
        _tc_final_body,
        grid=(GRID,),
        in_specs=[
            pl.BlockSpec((NC, BLK, 32), lambda i: (0, i, 0)),
            pl.BlockSpec((BLK, 32), lambda i: (i, 0)),
            pl.BlockSpec((BLK, 1), lambda i: (i, 0)),
            pl.BlockSpec((1, 32), lambda i: (0, 0)),
            pl.BlockSpec((32, 1), lambda i: (0, 0)),
            pl.BlockSpec((1, 1), lambda i: (0, 0)),
        ],
        out_specs=pl.BlockSpec((BLK, 1), lambda i: (i, 0)),
        out_shape=jax.ShapeDtypeStruct((N_NODES, 1), jnp.float32),
    )(aggp, g, d, b, Wl, bl)


def kernel(x, edge_index, W1, b1, W2, b2, W3, b3, Wl, bl):
    ei = edge_index.astype(jnp.int32)
    npad = E_PAD - N_EDGES
    pad_iota = lax.iota(jnp.int32, npad)
    src3 = jnp.concatenate([ei[0], pad_iota % N_NODES]).reshape(NW, NCHUNK, CH)
    dst3 = jnp.concatenate(
        [ei[1], N_NODES + pad_iota % (N_PAD - N_NODES)]
    ).reshape(NW, NCHUNK, CH)
    pad96 = lax.iota(jnp.int32, E_PAD96 - N_EDGES)
    src96 = jnp.concatenate([ei[0], pad96 % N_NODES]).reshape(NW, NCK96, C96)
    dst96 = jnp.concatenate(
        [ei[1], N_NODES + pad96 % (N_PAD - N_NODES)]
    ).reshape(NW, NCK96, C96)
    z16 = jnp.zeros((N_PAD, DEG_C), jnp.float32)

    h1x = _tc_mm(x, W1)
    degp = _deg_sc(dst3, z16)
    d, g1 = _tc_scale(degp, h1x)
    aggp1 = _agg128_96(src96, dst96, g1)
    g2 = _tc_mid_128_64(aggp1, g1, d, b1.reshape(1, -1), W2)
    aggp2 = _agg64(src3, dst3, g2)
    g3 = _tc_mid_64_32(aggp2, g2, d, b2.reshape(1, -1), W3)
    aggp3 = _agg32(src3, dst3, g3)
    return _tc_final(aggp3, g3, d, b3.reshape(1, -1), Wl, bl.reshape(1, 1))


# speedup vs baseline: 1.0773x; 1.0011x over previous
import functools

import jax
import jax.numpy as jnp
from jax import lax
from jax.experimental import pallas as pl
from jax.experimental.pallas import tpu as pltpu
from jax.experimental.pallas import tpu_sc as plsc

N_NODES = 10000
N_EDGES = 320000
NC = 2
NS = 16
NW = NC * NS
CH = 128
NCHUNK = 80
EPW = NCHUNK * CH
E_PAD = NW * EPW
N_PAD = 10240
RPS = N_PAD // NS
DEG_C = 16

_MESH = plsc.VectorSubcoreMesh(core_axis_name="c", subcore_axis_name="s")
_SC_PARAMS = pltpu.CompilerParams(use_tc_tiling_on_sc=False)


@functools.partial(
    pl.kernel,
    mesh=_MESH,
    compiler_params=_SC_PARAMS,
    out_type=jax.ShapeDtypeStruct((NC, N_PAD, DEG_C), jnp.float32),
    scratch_types=[
        pltpu.VMEM((NCHUNK, CH), jnp.int32),
        pltpu.VMEM((CH, DEG_C), jnp.float32),
        pltpu.VMEM_SHARED((N_PAD, DEG_C), jnp.float32),
        pltpu.SemaphoreType.DMA,
        pltpu.SemaphoreType.DMA,
        pltpu.SemaphoreType.DMA,
        pltpu.SemaphoreType.DMA,
    ],
)
def _deg_sc(dst_hbm, z_hbm, out_hbm, dst_v, ones_v, acc_sh, *ssems):
    cid = lax.axis_index("c")
    sid = lax.axis_index("s")
    wid = sid * NC + cid
    pltpu.sync_copy(z_hbm.at[pl.ds(sid * RPS, RPS)],
                    acc_sh.at[pl.ds(sid * RPS, RPS)])
    pltpu.sync_copy(dst_hbm.at[wid], dst_v)
    for i in range(CH):
        ones_v[i] = jnp.ones((DEG_C,), jnp.float32)
    plsc.subcore_barrier()

    def body(t, carry):
        for b in range(4):
            j = 4 * t + b

            @pl.when(t > 0)
            def _():
                pltpu.make_async_copy(ones_v, acc_sh.at[dst_v.at[j]],
                                      ssems[b]).wait()

            pltpu.async_copy(ones_v, acc_sh.at[dst_v.at[j]],
                             ssems[b], add=True)
        return carry

    lax.fori_loop(0, NCHUNK // 4, body, 0)
    for b in range(4):
        pltpu.make_async_copy(ones_v, acc_sh.at[dst_v.at[0]],
                              ssems[b]).wait()
    plsc.subcore_barrier()
    pltpu.sync_copy(acc_sh.at[pl.ds(sid * RPS, RPS)],
                    out_hbm.at[cid, pl.ds(sid * RPS, RPS)])


def _make_agg_sc(C, NBUF, FULL_IDX):
    NSTAGE = NCHUNK if FULL_IDX else NCHUNK // 2
    steps = NSTAGE // NBUF

    @functools.partial(
        pl.kernel,
        mesh=_MESH,
        compiler_params=_SC_PARAMS,
        out_type=jax.ShapeDtypeStruct((NC, N_PAD, C), jnp.float32),
        scratch_types=(
            [pltpu.VMEM((NSTAGE, CH), jnp.int32),
             pltpu.VMEM((NSTAGE, CH), jnp.int32)]
            + [pltpu.VMEM((CH, C), jnp.float32) for _ in range(NBUF)]
            + [pltpu.VMEM_SHARED((N_PAD, C), jnp.float32)]
            + [pltpu.SemaphoreType.DMA for _ in range(2 * NBUF + 1)]
        ),
    )
    def agg(src_hbm, dst_hbm, g_hbm, out_hbm, src_v, dst_v, *rest):
        rows = rest[:NBUF]
        acc_sh = rest[NBUF]
        gsems = rest[NBUF + 1:2 * NBUF + 1]
        ssems = rest[2 * NBUF + 1:3 * NBUF + 1]
        seedsem = rest[3 * NBUF + 1]
        cid = lax.axis_index("c")
        sid = lax.axis_index("s")
        wid = sid * NC + cid
        last = N_NODES - (NS - 1) * RPS

        @pl.when(sid < NS - 1)
        def _():
            pltpu.async_copy(g_hbm.at[pl.ds(sid * RPS, RPS)],
                             acc_sh.at[pl.ds(sid * RPS, RPS)], seedsem)

        @pl.when(sid == NS - 1)
        def _():
            pltpu.async_copy(g_hbm.at[pl.ds((NS - 1) * RPS, last)],
                             acc_sh.at[pl.ds((NS - 1) * RPS, last)], seedsem)

        pltpu.sync_copy(src_hbm.at[wid, pl.ds(0, NSTAGE)], src_v)
        pltpu.sync_copy(dst_hbm.at[wid, pl.ds(0, NSTAGE)], dst_v)

        @pl.when(sid < NS - 1)
        def _():
            pltpu.make_async_copy(g_hbm.at[pl.ds(sid * RPS, RPS)],
                                  acc_sh.at[pl.ds(sid * RPS, RPS)],
                                  seedsem).wait()

        @pl.when(sid == NS - 1)
        def _():
            pltpu.make_async_copy(g_hbm.at[pl.ds((NS - 1) * RPS, last)],
                                  acc_sh.at[pl.ds((NS - 1) * RPS, last)],
                                  seedsem).wait()

        plsc.subcore_barrier()

        for h in range(1 if FULL_IDX else 2):
            if h:
                pltpu.sync_copy(src_hbm.at[wid, pl.ds(NSTAGE, NSTAGE)], src_v)
                pltpu.sync_copy(dst_hbm.at[wid, pl.ds(NSTAGE, NSTAGE)], dst_v)
            for b in range(NBUF):
                pltpu.async_copy(g_hbm.at[src_v.at[b]], rows[b], gsems[b])

            def body(t, carry):
                for b in range(NBUF):
                    j = NBUF * t + b
                    pltpu.make_async_copy(g_hbm.at[src_v.at[j]], rows[b],
                                          gsems[b]).wait()
                    pltpu.async_copy(rows[b], acc_sh.at[dst_v.at[j]],
                                     ssems[b], add=True)

                    @pl.when(t < steps - 1)
                    def _():
                        pltpu.make_async_copy(rows[b],
                                              acc_sh.at[dst_v.at[j]],
                                              ssems[b]).wait()
                        pltpu.async_copy(g_hbm.at[src_v.at[j + NBUF]],
                                         rows[b], gsems[b])
                return carry

            lax.fori_loop(0, steps, body, 0)
            for b in range(NBUF):
                pltpu.make_async_copy(rows[b], acc_sh.at[dst_v.at[0]],
                                      ssems[b]).wait()
        plsc.subcore_barrier()

        @pl.when(sid < NS - 1)
        def _():
            pltpu.sync_copy(acc_sh.at[pl.ds(sid * RPS, RPS)],
                            out_hbm.at[cid, pl.ds(sid * RPS, RPS)])

        @pl.when(sid == NS - 1)
        def _():
            pltpu.sync_copy(acc_sh.at[pl.ds((NS - 1) * RPS, last)],
                            out_hbm.at[cid, pl.ds((NS - 1) * RPS, last)])

    return agg


C96 = 96
NCK96 = 108
E_PAD96 = NW * NCK96 * C96


def _make_agg128_96(NBUF=3):
    NSTAGE = NCK96 // 2
    steps = NSTAGE // NBUF

    @functools.partial(
        pl.kernel,
        mesh=_MESH,
        compiler_params=_SC_PARAMS,
        out_type=jax.ShapeDtypeStruct((NC, N_PAD, 128), jnp.float32),
        scratch_types=(
            [pltpu.VMEM((NSTAGE, C96), jnp.int32),
             pltpu.VMEM((NSTAGE, C96), jnp.int32)]
            + [pltpu.VMEM((C96, 128), jnp.float32) for _ in range(NBUF)]
            + [pltpu.VMEM_SHARED((N_PAD, 128), jnp.float32)]
            + [pltpu.SemaphoreType.DMA for _ in range(2 * NBUF + 1)]
        ),
    )
    def agg(src_hbm, dst_hbm, g_hbm, out_hbm, src_v, dst_v, *rest):
        rows = rest[:NBUF]
        acc_sh = rest[NBUF]
        gsems = rest[NBUF + 1:2 * NBUF + 1]
        ssems = rest[2 * NBUF + 1:3 * NBUF + 1]
        seedsem = rest[3 * NBUF + 1]
        cid = lax.axis_index("c")
        sid = lax.axis_index("s")
        wid = sid * NC + cid
        last = N_NODES - (NS - 1) * RPS

        @pl.when(sid < NS - 1)
        def _():
            pltpu.async_copy(g_hbm.at[pl.ds(sid * RPS, RPS)],
                             acc_sh.at[pl.ds(sid * RPS, RPS)], seedsem)

        @pl.when(sid == NS - 1)
        def _():
            pltpu.async_copy(g_hbm.at[pl.ds((NS - 1) * RPS, last)],
                             acc_sh.at[pl.ds((NS - 1) * RPS, last)], seedsem)

        pltpu.sync_copy(src_hbm.at[wid, pl.ds(0, NSTAGE)], src_v)
        pltpu.sync_copy(dst_hbm.at[wid, pl.ds(0, NSTAGE)], dst_v)

        @pl.when(sid < NS - 1)
        def _():
            pltpu.make_async_copy(g_hbm.at[pl.ds(sid * RPS, RPS)],
                                  acc_sh.at[pl.ds(sid * RPS, RPS)],
                                  seedsem).wait()

        @pl.when(sid == NS - 1)
        def _():
            pltpu.make_async_copy(g_hbm.at[pl.ds((NS - 1) * RPS, last)],
                                  acc_sh.at[pl.ds((NS - 1) * RPS, last)],
                                  seedsem).wait()

        plsc.subcore_barrier()

        for h in range(2):
            if h:
                pltpu.sync_copy(src_hbm.at[wid, pl.ds(NSTAGE, NSTAGE)], src_v)
                pltpu.sync_copy(dst_hbm.at[wid, pl.ds(NSTAGE, NSTAGE)], dst_v)
            for b in range(NBUF):
                pltpu.async_copy(g_hbm.at[src_v.at[b]], rows[b], gsems[b])

            def body(t, carry):
                for b in range(NBUF):
                    j = NBUF * t + b
                    pltpu.make_async_copy(g_hbm.at[src_v.at[j]], rows[b],
                                          gsems[b]).wait()
                    pltpu.async_copy(rows[b], acc_sh.at[dst_v.at[j]],
                                     ssems[b], add=True)

                    @pl.when(t < steps - 1)
                    def _():
                        pltpu.make_async_copy(rows[b],
                                              acc_sh.at[dst_v.at[j]],
                                              ssems[b]).wait()
                        pltpu.async_copy(g_hbm.at[src_v.at[j + NBUF]],
                                         rows[b], gsems[b])
                return carry

            lax.fori_loop(0, steps, body, 0)
            for b in range(NBUF):
                pltpu.make_async_copy(rows[b], acc_sh.at[dst_v.at[0]],
                                      ssems[b]).wait()
        plsc.subcore_barrier()

        @pl.when(sid < NS - 1)
        def _():
            pltpu.sync_copy(acc_sh.at[pl.ds(sid * RPS, RPS)],
                            out_hbm.at[cid, pl.ds(sid * RPS, RPS)])

        @pl.when(sid == NS - 1)
        def _():
            pltpu.sync_copy(acc_sh.at[pl.ds((NS - 1) * RPS, last)],
                            out_hbm.at[cid, pl.ds((NS - 1) * RPS, last)])

    return agg


_agg128_96 = _make_agg128_96()

_agg64 = _make_agg_sc(64, 4, True)
_agg32 = _make_agg_sc(32, 8, True)


BLK = 5000
GRID = N_NODES // BLK


def _tc_mm_body(x_ref, w_ref, out_ref):
    out_ref[...] = jnp.dot(x_ref[...], w_ref[...],
                           preferred_element_type=jnp.float32)


def _tc_mm(x, W1):
    return pl.pallas_call(
        _tc_mm_body,
        grid=(GRID,),
        in_specs=[
            pl.BlockSpec((BLK, 128), lambda i: (i, 0)),
            pl.BlockSpec((128, 128), lambda i: (0, 0)),
        ],
        out_specs=pl.BlockSpec((BLK, 128), lambda i: (i, 0)),
        out_shape=jax.ShapeDtypeStruct((N_NODES, 128), jnp.float32),
    )(x, W1)


def _tc_scale_body(degp_ref, h_ref, d_ref, g_ref):
    deg = degp_ref[0, :, 0:1] + degp_ref[1, :, 0:1] + 1.0
    d = lax.rsqrt(deg)
    d_ref[...] = d
    g_ref[...] = d * h_ref[...]


def _tc_scale(degp, h):
    return pl.pallas_call(
        _tc_scale_body,
        grid=(GRID,),
        in_specs=[
            pl.BlockSpec((NC, BLK, DEG_C), lambda i: (0, i, 0)),
            pl.BlockSpec((BLK, 128), lambda i: (i, 0)),
        ],
        out_specs=[
            pl.BlockSpec((BLK, 1), lambda i: (i, 0)),
            pl.BlockSpec((BLK, 128), lambda i: (i, 0)),
        ],
        out_shape=[
            jax.ShapeDtypeStruct((N_NODES, 1), jnp.float32),
            jax.ShapeDtypeStruct((N_NODES, 128), jnp.float32),
        ],
    )(degp, h)


def _tc_mid_body(aggp_ref, g_ref, d_ref, b_ref, w_ref, out_ref):
    d = d_ref[...]
    h = jnp.maximum(d * (aggp_ref[0] + aggp_ref[1] - g_ref[...]) + b_ref[...],
                    0.0)
    out_ref[...] = d * jnp.dot(h, w_ref[...],
                               preferred_element_type=jnp.float32)


def _make_tc_mid(Cin, Cout):
    def run(aggp, g, d, b, W):
        return pl.pallas_call(
            _tc_mid_body,
            grid=(GRID,),
            in_specs=[
                pl.BlockSpec((NC, BLK, Cin), lambda i: (0, i, 0)),
                pl.BlockSpec((BLK, Cin), lambda i: (i, 0)),
                pl.BlockSpec((BLK, 1), lambda i: (i, 0)),
                pl.BlockSpec((1, Cin), lambda i: (0, 0)),
                pl.BlockSpec((Cin, Cout), lambda i: (0, 0)),
            ],
            out_specs=pl.BlockSpec((BLK, Cout), lambda i: (i, 0)),
            out_shape=jax.ShapeDtypeStruct((N_NODES, Cout), jnp.float32),
        )(aggp, g, d, b, W)

    return run


_tc_mid_128_64 = _make_tc_mid(128, 64)
_tc_mid_64_32 = _make_tc_mid(64, 32)


def _tc_final_body(aggp_ref, g_ref, d_ref, b_ref, wl_ref, bl_ref, out_ref):
    d = d_ref[...]
    h = jnp.maximum(d * (aggp_ref[0] + aggp_ref[1] - g_ref[...]) + b_ref[...],
                    0.0)
    out_ref[...] = jnp.dot(h, wl_ref[...],
                           preferred_element_type=jnp.float32) + bl_ref[...]


def _tc_final(aggp, g, d, b, Wl, bl):
    return pl.pallas_call(
        _tc_final_body,
        grid=(GRID,),
        in_specs=[
            pl.BlockSpec((NC, BLK, 32), lambda i: (0, i, 0)),
            pl.BlockSpec((BLK, 32), lambda i: (i, 0)),
            pl.BlockSpec((BLK, 1), lambda i: (i, 0)),
            pl.BlockSpec((1, 32), lambda i: (0, 0)),
            pl.BlockSpec((32, 1), lambda i: (0, 0)),
            pl.BlockSpec((1, 1), lambda i: (0, 0)),
        ],
        out_specs=pl.BlockSpec((BLK, 1), lambda i: (i, 0)),
        out_shape=jax.ShapeDtypeStruct((N_NODES, 1), jnp.float32),
    )(aggp, g, d, b, Wl, bl)


def kernel(x, edge_index, W1, b1, W2, b2, W3, b3, Wl, bl):
    ei = edge_index.astype(jnp.int32)
    npad = E_PAD - N_EDGES
    pad_iota = lax.iota(jnp.int32, npad)
    src3 = jnp.concatenate([ei[0], pad_iota % N_NODES]).reshape(NW, NCHUNK, CH)
    dst3 = jnp.concatenate(
        [ei[1], N_NODES + pad_iota % (N_PAD - N_NODES)]
    ).reshape(NW, NCHUNK, CH)
    pad96 = lax.iota(jnp.int32, E_PAD96 - N_EDGES)
    src96 = jnp.concatenate([ei[0], pad96 % N_NODES]).reshape(NW, NCK96, C96)
    dst96 = jnp.concatenate(
        [ei[1], N_NODES + pad96 % (N_PAD - N_NODES)]
    ).reshape(NW, NCK96, C96)
    z16 = jnp.zeros((N_PAD, DEG_C), jnp.float32)

    h1x = _tc_mm(x, W1)
    degp = _deg_sc(dst3, z16)
    d, g1 = _tc_scale(degp, h1x)
    aggp1 = _agg128_96(src96, dst96, g1)
    g2 = _tc_mid_128_64(aggp1, g1, d, b1.reshape(1, -1), W2)
    aggp2 = _agg64(src3, dst3, g2)
    g3 = _tc_mid_64_32(aggp2, g2, d, b2.reshape(1, -1), W3)
    aggp3 = _agg32(src3, dst3, g3)
    return _tc_final(aggp3, g3, d, b3.reshape(1, -1), Wl, bl.reshape(1, 1))
